# baseline (speedup 1.0000x reference)
import functools

import jax
import jax.numpy as jnp
from jax import lax
from jax.experimental import pallas as pl
from jax.experimental.pallas import tpu as pltpu
from jax.experimental.pallas import tpu_sc as plsc

VOCAB = 1000000
DIM = 16
BATCH = 4096
CTX_LEN = 200
Q_LEN = 20

NW = 32
CH = 1024
NBT = CH // 128
PLANE = NBT * 1024

OC_LEN = CTX_LEN * 2 * 96 * 8 * 128
OQ_LEN = Q_LEN * 2 * 96 * 8 * 128

_mesh = plsc.VectorSubcoreMesh(core_axis_name="c", subcore_axis_name="s")


@functools.partial(
    pl.kernel,
    mesh=_mesh,
    out_type=[
        jax.ShapeDtypeStruct((OC_LEN,), jnp.float32),
        jax.ShapeDtypeStruct((OQ_LEN,), jnp.float32),
    ],
    scratch_types=[
        pltpu.VMEM((BATCH,), jnp.int32),
        pltpu.VMEM((BATCH,), jnp.int32),
        [pltpu.VMEM((CH, DIM), jnp.float32) for _ in range(4)],
        [pltpu.VMEM((2 * PLANE,), jnp.float32) for _ in range(2)],
        [pltpu.SemaphoreType.DMA for _ in range(4)],
        [pltpu.SemaphoreType.DMA for _ in range(2)],
        [pltpu.SemaphoreType.DMA for _ in range(2)],
    ],
    compiler_params=pltpu.CompilerParams(
        use_tc_tiling_on_sc=False, needs_layout_passes=False
    ),
)
def _embed(w0, w1, w2, ctxf, qf, oc, oq, idxA, idxB, rows, trs, gsem, isem, wsem):
    wid = lax.axis_index("s") * 2 + lax.axis_index("c")
    lane = lax.iota(jnp.int32, 16)
    pat = lane * 128
    pat2 = pat + jnp.where(lane >= 8, PLANE - 1024, 0)

    def transpose_chunk(rv, p2):
        @plsc.parallel_loop(0, CH, step=1, unroll=8)
        def _(i):
            row = rv[i, :]
            off = (i >> 7) * 1024 + (i & 127)
            plsc.store_scatter(p2, [pat2 + off], row)

    def fire_gather(w, idx_ref, b):
        pltpu.async_copy(w.at[idx_ref.at[pl.ds(b * CH, CH)]], rows[b], gsem[b])

    def wait_gather(w, b):
        pltpu.make_async_copy(w.at[idxA.at[pl.ds(0, CH)]], rows[b], gsem[b]).wait()

    def fire_writes(dst, s, t, b, pr):
        c0 = t * 32 + b * NBT
        dst0 = dst.at[pl.ds(((s * 2 + 0) * 96 + c0) * 1024, PLANE)]
        dst1 = dst.at[pl.ds(((s * 2 + 1) * 96 + c0) * 1024, PLANE)]
        pltpu.async_copy(trs[pr].at[pl.ds(0, PLANE)], dst0, wsem[pr])
        pltpu.async_copy(trs[pr].at[pl.ds(PLANE, PLANE)], dst1, wsem[pr])

    def wait_writes(dst, pr):
        pltpu.make_async_copy(trs[pr].at[pl.ds(0, PLANE)], dst.at[pl.ds(0, PLANE)], wsem[pr]).wait()
        pltpu.make_async_copy(trs[pr].at[pl.ds(0, PLANE)], dst.at[pl.ds(0, PLANE)], wsem[pr]).wait()

    def fire_idx(src, s, idx_ref, pr):
        pltpu.async_copy(src.at[pl.ds(s * BATCH, BATCH)], idx_ref, isem[pr])

    def wait_idx(src, idx_ref, pr):
        pltpu.make_async_copy(src.at[pl.ds(0, BATCH)], idx_ref, isem[pr]).wait()

    for t, w in enumerate((w0, w1, w2)):
        r0 = (wid + 11 * t) % 32

        pltpu.sync_copy(ctxf.at[pl.ds(r0 * BATCH, BATCH)], idxA)
        for b in range(4):
            fire_gather(w, idxA, b)
        fire_idx(ctxf, r0 + NW, idxB, 1)

        def section(g, idx_cur, pr_cur, idx_nxt, pr_nxt, w=w, t=t, r0=r0):
            s = r0 + NW * g

            @pl.when(s < CTX_LEN)
            def _():
                s_n1 = s + NW
                s_n2 = s + 2 * NW
                for b in range(4):
                    wait_gather(w, b)
                    if b < 2:
                        @pl.when(g > 0)
                        def _():
                            wait_writes(oc, b % 2)
                    else:
                        wait_writes(oc, b % 2)
                    transpose_chunk(rows[b], trs[b % 2])
                    fire_writes(oc, s, t, b, b % 2)
                    if b == 0:
                        @pl.when(s_n1 < CTX_LEN)
                        def _():
                            wait_idx(ctxf, idx_nxt, pr_nxt)
                    if b == 3:
                        @pl.when(s_n2 < CTX_LEN)
                        def _():
                            fire_idx(ctxf, s_n2, idx_cur, pr_cur)

                    @pl.when(s_n1 < CTX_LEN)
                    def _(b=b):
                        fire_gather(w, idx_nxt, b)

        def pair_body(jj, carry):
            section(2 * jj, idxA, 0, idxB, 1)
            section(2 * jj + 1, idxB, 1, idxA, 0)
            return carry

        lax.fori_loop(0, 4, pair_body, 0)
        wait_writes(oc, 0)
        wait_writes(oc, 1)

        rq = (wid + 7 * t) % 32

        @pl.when(rq < Q_LEN)
        def _(w=w, t=t, rq=rq):
            pltpu.sync_copy(qf.at[pl.ds(rq * BATCH, BATCH)], idxA)
            for b in range(4):
                fire_gather(w, idxA, b)
            for b in range(4):
                wait_gather(w, b)
                if b >= 2:
                    wait_writes(oq, b % 2)
                transpose_chunk(rows[b], trs[b % 2])
                fire_writes(oq, rq, t, b, b % 2)
            wait_writes(oq, 0)
            wait_writes(oq, 1)


KC = 1024
NFULL = VOCAB // KC
REM_C0 = NFULL * KC
TAIL_C0 = REM_C0 + 512


@functools.partial(
    pl.kernel,
    mesh=_mesh,
    out_type=[jax.ShapeDtypeStruct((VOCAB // 8, 128), jnp.float32) for _ in range(3)],
    scratch_types=[
        [pltpu.VMEM((16, KC), jnp.float32) for _ in range(2)],
        [pltpu.VMEM((128, 128), jnp.float32) for _ in range(2)],
        [pltpu.SemaphoreType.DMA for _ in range(2)],
        [pltpu.SemaphoreType.DMA for _ in range(2)],
    ],
    compiler_params=pltpu.CompilerParams(
        use_tc_tiling_on_sc=True, needs_layout_passes=False
    ),
)
def _compact(w0t, w1t, w2t, tl0, tl1, tl2, o0, o1, o2, tins, touts, rsem, wsem):
    wid = lax.axis_index("s") * 2 + lax.axis_index("c")
    lane = lax.iota(jnp.int32, 16)

    def transpose_block(tin, tout, ncols):
        @plsc.parallel_loop(0, ncols, step=1, unroll=8)
        def _(i):
            zero = lane - lane
            col = plsc.load_gather(tin, [lane, zero + i])
            tout[i >> 3, pl.ds((i & 7) * 16, 16)] = col

    def fire_read(w, e, p):
        pltpu.async_copy(w.at[:, pl.ds(e * KC, KC)], tins[p], rsem[p])

    def wait_read(w, p):
        pltpu.make_async_copy(w.at[:, pl.ds(0, KC)], tins[p], rsem[p]).wait()

    def fire_write(dst, e, p):
        pltpu.async_copy(touts[p], dst.at[pl.ds(e * 128, 128)], wsem[p])

    def wait_write(dst, p):
        pltpu.make_async_copy(touts[p], dst.at[pl.ds(0, 128)], wsem[p]).wait()

    for w, tl, dst in ((w0t, tl0, o0), (w1t, tl1, o1), (w2t, tl2, o2)):
        fire_read(w, wid, 0)
        fire_read(w, wid + NW, 1)

        def c1_section(k, jj, p, w=w, dst=dst):
            e = wid + NW * k

            @pl.when(e < NFULL)
            def _():
                wait_read(w, p)

                @pl.when(jj > 0)
                def _():
                    wait_write(dst, p)

                transpose_block(tins[p], touts[p], KC)
                fire_write(dst, e, p)

                @pl.when(e + 2 * NW < NFULL)
                def _():
                    fire_read(w, e + 2 * NW, p)

        def c1_pair(jj, carry, w=w, dst=dst):
            c1_section(2 * jj, jj, 0)
            c1_section(2 * jj + 1, jj, 1)
            return carry

        lax.fori_loop(0, 16, c1_pair, 0)
        wait_write(dst, 0)
        wait_write(dst, 1)

        @pl.when(wid == 31)
        def _(w=w, dst=dst):
            pltpu.sync_copy(w.at[:, pl.ds(REM_C0, 512)], tins[0].at[:, pl.ds(0, 512)])
            transpose_block(tins[0], touts[0], 512)
            pltpu.sync_copy(touts[0].at[pl.ds(0, 64)], dst.at[pl.ds(REM_C0 // 8, 64)])

        @pl.when(wid == 30)
        def _(tl=tl, dst=dst):
            pltpu.sync_copy(tl, dst.at[pl.ds(TAIL_C0 // 8, 8)])


def kernel(W_word, W_pos, W_kg, context, question):
    ctxf = context.T.reshape(-1)
    qf = question.T.reshape(-1)
    c0, c1, c2 = _compact(
        W_word.T, W_pos.T, W_kg.T,
        W_word[TAIL_C0:].reshape(8, 128),
        W_pos[TAIL_C0:].reshape(8, 128),
        W_kg[TAIL_C0:].reshape(8, 128),
    )
    oc, oq = _embed(
        c0.reshape(VOCAB, DIM), c1.reshape(VOCAB, DIM), c2.reshape(VOCAB, DIM),
        ctxf, qf,
    )
    out_c = (
        oc.reshape(CTX_LEN, 2, 96, 8, 128)
        .transpose(2, 4, 0, 1, 3)
        .reshape(3 * BATCH, CTX_LEN, DIM)
    )
    out_q = (
        oq.reshape(Q_LEN, 2, 96, 8, 128)
        .transpose(2, 4, 0, 1, 3)
        .reshape(3 * BATCH, Q_LEN, DIM)
    )
    return out_c, out_q


# speedup vs baseline: 1.0381x; 1.0381x over previous
import functools

import jax
import jax.numpy as jnp
from jax import lax
from jax.experimental import pallas as pl
from jax.experimental.pallas import tpu as pltpu
from jax.experimental.pallas import tpu_sc as plsc

VOCAB = 1000000
DIM = 16
BATCH = 4096
CTX_LEN = 200
Q_LEN = 20

NW = 32
CH = 1024
NBT = CH // 128
PLANE = NBT * 1024

OC_LEN = CTX_LEN * 2 * 96 * 8 * 128
OQ_LEN = Q_LEN * 2 * 96 * 8 * 128

_mesh = plsc.VectorSubcoreMesh(core_axis_name="c", subcore_axis_name="s")


@functools.partial(
    pl.kernel,
    mesh=_mesh,
    out_type=[
        jax.ShapeDtypeStruct((OC_LEN,), jnp.float32),
        jax.ShapeDtypeStruct((OQ_LEN,), jnp.float32),
    ],
    scratch_types=[
        pltpu.VMEM((BATCH,), jnp.int32),
        pltpu.VMEM((BATCH,), jnp.int32),
        [pltpu.VMEM((CH, DIM), jnp.float32) for _ in range(4)],
        [pltpu.VMEM((PLANE,), jnp.float32) for _ in range(4)],
        [pltpu.SemaphoreType.DMA for _ in range(4)],
        [pltpu.SemaphoreType.DMA for _ in range(2)],
        [pltpu.SemaphoreType.DMA for _ in range(2)],
    ],
    compiler_params=pltpu.CompilerParams(
        use_tc_tiling_on_sc=False, needs_layout_passes=False
    ),
)
def _embed(w0, w1, w2, ctxf, qf, oc, oq, idxA, idxB, rows, trs, gsem, isem, wsem):
    wid = lax.axis_index("s") * 2 + lax.axis_index("c")
    lane = lax.iota(jnp.int32, 16)
    pat = lane * 128
    m_lo = lane < 8
    m_hi = lane >= 8

    pat1 = pat - 1024

    def transpose_chunk(rv, p0, p1):
        @plsc.parallel_loop(0, CH, step=1, unroll=16)
        def _(i):
            row = rv[i, :]
            off = (i >> 7) * 1024 + (i & 127)
            plsc.store_scatter(p0, [pat + off], row, mask=m_lo)
            plsc.store_scatter(p1, [pat1 + off], row, mask=m_hi)

    def fire_gather(w, idx_ref, b):
        pltpu.async_copy(w.at[idx_ref.at[pl.ds(b * CH, CH)]], rows[b], gsem[b])

    def wait_gather(w, b):
        pltpu.make_async_copy(w.at[idxA.at[pl.ds(0, CH)]], rows[b], gsem[b]).wait()

    def fire_writes(dst, s, t, b, pr):
        c0 = t * 32 + b * NBT
        dst0 = dst.at[pl.ds(((s * 2 + 0) * 96 + c0) * 1024, PLANE)]
        dst1 = dst.at[pl.ds(((s * 2 + 1) * 96 + c0) * 1024, PLANE)]
        pltpu.async_copy(trs[2 * pr], dst0, wsem[pr])
        pltpu.async_copy(trs[2 * pr + 1], dst1, wsem[pr])

    def wait_writes(dst, pr):
        pltpu.make_async_copy(trs[2 * pr], dst.at[pl.ds(0, PLANE)], wsem[pr]).wait()
        pltpu.make_async_copy(trs[2 * pr + 1], dst.at[pl.ds(0, PLANE)], wsem[pr]).wait()

    def fire_idx(src, s, idx_ref, pr):
        pltpu.async_copy(src.at[pl.ds(s * BATCH, BATCH)], idx_ref, isem[pr])

    def wait_idx(src, idx_ref, pr):
        pltpu.make_async_copy(src.at[pl.ds(0, BATCH)], idx_ref, isem[pr]).wait()

    for t, w in enumerate((w0, w1, w2)):
        r0 = (wid + 11 * t) % 32

        pltpu.sync_copy(ctxf.at[pl.ds(r0 * BATCH, BATCH)], idxA)
        for b in range(4):
            fire_gather(w, idxA, b)
        fire_idx(ctxf, r0 + NW, idxB, 1)

        def section(g, idx_cur, pr_cur, idx_nxt, pr_nxt, w=w, t=t, r0=r0):
            s = r0 + NW * g

            @pl.when(s < CTX_LEN)
            def _():
                s_n1 = s + NW
                s_n2 = s + 2 * NW
                for b in range(4):
                    wait_gather(w, b)
                    if b < 2:
                        @pl.when(g > 0)
                        def _():
                            wait_writes(oc, b % 2)
                    else:
                        wait_writes(oc, b % 2)
                    transpose_chunk(rows[b], trs[2 * (b % 2)], trs[2 * (b % 2) + 1])
                    fire_writes(oc, s, t, b, b % 2)
                    if b == 0:
                        @pl.when(s_n1 < CTX_LEN)
                        def _():
                            wait_idx(ctxf, idx_nxt, pr_nxt)
                    if b == 3:
                        @pl.when(s_n2 < CTX_LEN)
                        def _():
                            fire_idx(ctxf, s_n2, idx_cur, pr_cur)

                    @pl.when(s_n1 < CTX_LEN)
                    def _(b=b):
                        fire_gather(w, idx_nxt, b)

        def pair_body(jj, carry):
            section(2 * jj, idxA, 0, idxB, 1)
            section(2 * jj + 1, idxB, 1, idxA, 0)
            return carry

        lax.fori_loop(0, 4, pair_body, 0)
        wait_writes(oc, 0)
        wait_writes(oc, 1)

        rq = (wid + 7 * t) % 32

        @pl.when(rq < Q_LEN)
        def _(w=w, t=t, rq=rq):
            pltpu.sync_copy(qf.at[pl.ds(rq * BATCH, BATCH)], idxA)
            for b in range(4):
                fire_gather(w, idxA, b)
            for b in range(4):
                wait_gather(w, b)
                if b >= 2:
                    wait_writes(oq, b % 2)
                transpose_chunk(rows[b], trs[2 * (b % 2)], trs[2 * (b % 2) + 1])
                fire_writes(oq, rq, t, b, b % 2)
            wait_writes(oq, 0)
            wait_writes(oq, 1)


KC = 1024
NFULL = VOCAB // KC
REM_C0 = NFULL * KC
TAIL_C0 = REM_C0 + 512


@functools.partial(
    pl.kernel,
    mesh=_mesh,
    out_type=[jax.ShapeDtypeStruct((VOCAB // 8, 128), jnp.float32) for _ in range(3)],
    scratch_types=[
        [pltpu.VMEM((16, KC), jnp.float32) for _ in range(2)],
        [pltpu.VMEM((128, 128), jnp.float32) for _ in range(2)],
        [pltpu.SemaphoreType.DMA for _ in range(2)],
        [pltpu.SemaphoreType.DMA for _ in range(2)],
    ],
    compiler_params=pltpu.CompilerParams(
        use_tc_tiling_on_sc=True, needs_layout_passes=False
    ),
)
def _compact(w0t, w1t, w2t, tl0, tl1, tl2, o0, o1, o2, tins, touts, rsem, wsem):
    wid = lax.axis_index("s") * 2 + lax.axis_index("c")
    lane = lax.iota(jnp.int32, 16)

    def transpose_block(tin, tout, ncols):
        @plsc.parallel_loop(0, ncols, step=1, unroll=16)
        def _(i):
            zero = lane - lane
            col = plsc.load_gather(tin, [lane, zero + i])
            tout[i >> 3, pl.ds((i & 7) * 16, 16)] = col

    def fire_read(w, e, p):
        pltpu.async_copy(w.at[:, pl.ds(e * KC, KC)], tins[p], rsem[p])

    def wait_read(w, p):
        pltpu.make_async_copy(w.at[:, pl.ds(0, KC)], tins[p], rsem[p]).wait()

    def fire_write(dst, e, p):
        pltpu.async_copy(touts[p], dst.at[pl.ds(e * 128, 128)], wsem[p])

    def wait_write(dst, p):
        pltpu.make_async_copy(touts[p], dst.at[pl.ds(0, 128)], wsem[p]).wait()

    for w, tl, dst in ((w0t, tl0, o0), (w1t, tl1, o1), (w2t, tl2, o2)):
        fire_read(w, wid, 0)
        fire_read(w, wid + NW, 1)

        def c1_section(k, jj, p, w=w, dst=dst):
            e = wid + NW * k

            @pl.when(e < NFULL)
            def _():
                wait_read(w, p)

                @pl.when(jj > 0)
                def _():
                    wait_write(dst, p)

                transpose_block(tins[p], touts[p], KC)
                fire_write(dst, e, p)

                @pl.when(e + 2 * NW < NFULL)
                def _():
                    fire_read(w, e + 2 * NW, p)

        def c1_pair(jj, carry, w=w, dst=dst):
            c1_section(2 * jj, jj, 0)
            c1_section(2 * jj + 1, jj, 1)
            return carry

        lax.fori_loop(0, 16, c1_pair, 0)
        wait_write(dst, 0)
        wait_write(dst, 1)

        @pl.when(wid == 31)
        def _(w=w, dst=dst):
            pltpu.sync_copy(w.at[:, pl.ds(REM_C0, 512)], tins[0].at[:, pl.ds(0, 512)])
            transpose_block(tins[0], touts[0], 512)
            pltpu.sync_copy(touts[0].at[pl.ds(0, 64)], dst.at[pl.ds(REM_C0 // 8, 64)])

        @pl.when(wid == 30)
        def _(tl=tl, dst=dst):
            pltpu.sync_copy(tl, dst.at[pl.ds(TAIL_C0 // 8, 8)])


def kernel(W_word, W_pos, W_kg, context, question):
    ctxf = context.T.reshape(-1)
    qf = question.T.reshape(-1)
    c0, c1, c2 = _compact(
        W_word.T, W_pos.T, W_kg.T,
        W_word[TAIL_C0:].reshape(8, 128),
        W_pos[TAIL_C0:].reshape(8, 128),
        W_kg[TAIL_C0:].reshape(8, 128),
    )
    oc, oq = _embed(
        c0.reshape(VOCAB, DIM), c1.reshape(VOCAB, DIM), c2.reshape(VOCAB, DIM),
        ctxf, qf,
    )
    out_c = (
        oc.reshape(CTX_LEN, 2, 96, 8, 128)
        .transpose(2, 4, 0, 1, 3)
        .reshape(3 * BATCH, CTX_LEN, DIM)
    )
    out_q = (
        oq.reshape(Q_LEN, 2, 96, 8, 128)
        .transpose(2, 4, 0, 1, 3)
        .reshape(3 * BATCH, Q_LEN, DIM)
    )
    return out_c, out_q


# speedup vs baseline: 1.3622x; 1.3123x over previous
import functools

import jax
import jax.numpy as jnp
from jax import lax
from jax.experimental import pallas as pl
from jax.experimental.pallas import tpu as pltpu
from jax.experimental.pallas import tpu_sc as plsc

VOCAB = 1000000
DIM = 16
BATCH = 4096
CTX_LEN = 200
Q_LEN = 20

NW = 32
CH = 1024
NBT = CH // 128
PLANE = NBT * 1024

OC_LEN = CTX_LEN * 2 * 96 * 8 * 128
OQ_LEN = Q_LEN * 2 * 96 * 8 * 128

_mesh = plsc.VectorSubcoreMesh(core_axis_name="c", subcore_axis_name="s")


@functools.partial(
    pl.kernel,
    mesh=_mesh,
    out_type=[
        jax.ShapeDtypeStruct((OC_LEN,), jnp.float32),
        jax.ShapeDtypeStruct((OQ_LEN,), jnp.float32),
    ],
    scratch_types=[
        pltpu.VMEM((BATCH,), jnp.int32),
        pltpu.VMEM((BATCH,), jnp.int32),
        [pltpu.VMEM((CH, DIM), jnp.float32) for _ in range(4)],
        [pltpu.VMEM((PLANE,), jnp.float32) for _ in range(4)],
        [pltpu.SemaphoreType.DMA for _ in range(4)],
        [pltpu.SemaphoreType.DMA for _ in range(2)],
        [pltpu.SemaphoreType.DMA for _ in range(2)],
    ],
    compiler_params=pltpu.CompilerParams(
        use_tc_tiling_on_sc=False, needs_layout_passes=False
    ),
)
def _embed(w0, w1, w2, ctxf, qf, oc, oq, idxA, idxB, rows, trs, gsem, isem, wsem):
    wid = lax.axis_index("s") * 2 + lax.axis_index("c")
    lane = lax.iota(jnp.int32, 16)
    pat = lane * 128
    m_lo = lane < 8
    m_hi = lane >= 8

    pat1 = pat - 1024

    def transpose_chunk(rv, p0, p1):
        @plsc.parallel_loop(0, CH, step=1, unroll=32)
        def _(i):
            row = rv[i, :]
            off = (i >> 7) * 1024 + (i & 127)
            plsc.store_scatter(p0, [pat + off], row, mask=m_lo)
            plsc.store_scatter(p1, [pat1 + off], row, mask=m_hi)

    def fire_gather(w, idx_ref, b):
        pltpu.async_copy(w.at[idx_ref.at[pl.ds(b * CH, CH)]], rows[b], gsem[b])

    def wait_gather(w, b):
        pltpu.make_async_copy(w.at[idxA.at[pl.ds(0, CH)]], rows[b], gsem[b]).wait()

    def fire_writes(dst, s, t, b, pr):
        c0 = t * 32 + b * NBT
        dst0 = dst.at[pl.ds(((s * 2 + 0) * 96 + c0) * 1024, PLANE)]
        dst1 = dst.at[pl.ds(((s * 2 + 1) * 96 + c0) * 1024, PLANE)]
        pltpu.async_copy(trs[2 * pr], dst0, wsem[pr])
        pltpu.async_copy(trs[2 * pr + 1], dst1, wsem[pr])

    def wait_writes(dst, pr):
        pltpu.make_async_copy(trs[2 * pr], dst.at[pl.ds(0, PLANE)], wsem[pr]).wait()
        pltpu.make_async_copy(trs[2 * pr + 1], dst.at[pl.ds(0, PLANE)], wsem[pr]).wait()

    def fire_idx(src, s, idx_ref, pr):
        pltpu.async_copy(src.at[pl.ds(s * BATCH, BATCH)], idx_ref, isem[pr])

    def wait_idx(src, idx_ref, pr):
        pltpu.make_async_copy(src.at[pl.ds(0, BATCH)], idx_ref, isem[pr]).wait()

    for t, w in enumerate((w0, w1, w2)):
        r0 = (wid + 11 * t) % 32

        pltpu.sync_copy(ctxf.at[pl.ds(r0 * BATCH, BATCH)], idxA)
        for b in range(4):
            fire_gather(w, idxA, b)
        fire_idx(ctxf, r0 + NW, idxB, 1)

        def section(g, idx_cur, pr_cur, idx_nxt, pr_nxt, w=w, t=t, r0=r0):
            s = r0 + NW * g

            @pl.when(s < CTX_LEN)
            def _():
                s_n1 = s + NW
                s_n2 = s + 2 * NW
                for b in range(4):
                    wait_gather(w, b)
                    if b < 2:
                        @pl.when(g > 0)
                        def _():
                            wait_writes(oc, b % 2)
                    else:
                        wait_writes(oc, b % 2)
                    transpose_chunk(rows[b], trs[2 * (b % 2)], trs[2 * (b % 2) + 1])
                    fire_writes(oc, s, t, b, b % 2)
                    if b == 0:
                        @pl.when(s_n1 < CTX_LEN)
                        def _():
                            wait_idx(ctxf, idx_nxt, pr_nxt)
                    if b == 3:
                        @pl.when(s_n2 < CTX_LEN)
                        def _():
                            fire_idx(ctxf, s_n2, idx_cur, pr_cur)

                    @pl.when(s_n1 < CTX_LEN)
                    def _(b=b):
                        fire_gather(w, idx_nxt, b)

        def pair_body(jj, carry):
            section(2 * jj, idxA, 0, idxB, 1)
            section(2 * jj + 1, idxB, 1, idxA, 0)
            return carry

        lax.fori_loop(0, 4, pair_body, 0)
        wait_writes(oc, 0)
        wait_writes(oc, 1)

        rq = (wid + 7 * t) % 32

        @pl.when(rq < Q_LEN)
        def _(w=w, t=t, rq=rq):
            pltpu.sync_copy(qf.at[pl.ds(rq * BATCH, BATCH)], idxA)
            for b in range(4):
                fire_gather(w, idxA, b)
            for b in range(4):
                wait_gather(w, b)
                if b >= 2:
                    wait_writes(oq, b % 2)
                transpose_chunk(rows[b], trs[2 * (b % 2)], trs[2 * (b % 2) + 1])
                fire_writes(oq, rq, t, b, b % 2)
            wait_writes(oq, 0)
            wait_writes(oq, 1)


KC = 1024
NFULL = VOCAB // KC
REM_C0 = NFULL * KC
TAIL_C0 = REM_C0 + 512


@functools.partial(
    pl.kernel,
    mesh=_mesh,
    out_type=[jax.ShapeDtypeStruct((VOCAB // 8, 128), jnp.float32) for _ in range(3)],
    scratch_types=[
        [pltpu.VMEM((16, KC), jnp.float32) for _ in range(2)],
        [pltpu.VMEM((128, 128), jnp.float32) for _ in range(2)],
        [pltpu.SemaphoreType.DMA for _ in range(2)],
        [pltpu.SemaphoreType.DMA for _ in range(2)],
    ],
    compiler_params=pltpu.CompilerParams(
        use_tc_tiling_on_sc=True, needs_layout_passes=False
    ),
)
def _compact(w0t, w1t, w2t, tl0, tl1, tl2, o0, o1, o2, tins, touts, rsem, wsem):
    wid = lax.axis_index("s") * 2 + lax.axis_index("c")
    lane = lax.iota(jnp.int32, 16)

    def transpose_block(tin, tout, ncols):
        @plsc.parallel_loop(0, ncols, step=1, unroll=32)
        def _(i):
            zero = lane - lane
            col = plsc.load_gather(tin, [lane, zero + i])
            tout[i >> 3, pl.ds((i & 7) * 16, 16)] = col

    def fire_read(w, e, p):
        pltpu.async_copy(w.at[:, pl.ds(e * KC, KC)], tins[p], rsem[p])

    def wait_read(w, p):
        pltpu.make_async_copy(w.at[:, pl.ds(0, KC)], tins[p], rsem[p]).wait()

    def fire_write(dst, e, p):
        pltpu.async_copy(touts[p], dst.at[pl.ds(e * 128, 128)], wsem[p])

    def wait_write(dst, p):
        pltpu.make_async_copy(touts[p], dst.at[pl.ds(0, 128)], wsem[p]).wait()

    for w, tl, dst in ((w0t, tl0, o0), (w1t, tl1, o1), (w2t, tl2, o2)):
        fire_read(w, wid, 0)
        fire_read(w, wid + NW, 1)

        def c1_section(k, jj, p, w=w, dst=dst):
            e = wid + NW * k

            @pl.when(e < NFULL)
            def _():
                wait_read(w, p)

                @pl.when(jj > 0)
                def _():
                    wait_write(dst, p)

                transpose_block(tins[p], touts[p], KC)
                fire_write(dst, e, p)

                @pl.when(e + 2 * NW < NFULL)
                def _():
                    fire_read(w, e + 2 * NW, p)

        def c1_pair(jj, carry, w=w, dst=dst):
            c1_section(2 * jj, jj, 0)
            c1_section(2 * jj + 1, jj, 1)
            return carry

        lax.fori_loop(0, 16, c1_pair, 0)
        wait_write(dst, 0)
        wait_write(dst, 1)

        @pl.when(wid == 31)
        def _(w=w, dst=dst):
            pltpu.sync_copy(w.at[:, pl.ds(REM_C0, 512)], tins[0].at[:, pl.ds(0, 512)])
            transpose_block(tins[0], touts[0], 512)
            pltpu.sync_copy(touts[0].at[pl.ds(0, 64)], dst.at[pl.ds(REM_C0 // 8, 64)])

        @pl.when(wid == 30)
        def _(tl=tl, dst=dst):
            pltpu.sync_copy(tl, dst.at[pl.ds(TAIL_C0 // 8, 8)])


def kernel(W_word, W_pos, W_kg, context, question):
    ctxf = context.T.reshape(-1)
    qf = question.T.reshape(-1)
    c0, c1, c2 = _compact(
        W_word.T, W_pos.T, W_kg.T,
        W_word[TAIL_C0:].reshape(8, 128),
        W_pos[TAIL_C0:].reshape(8, 128),
        W_kg[TAIL_C0:].reshape(8, 128),
    )
    oc, oq = _embed(
        c0.reshape(VOCAB, DIM), c1.reshape(VOCAB, DIM), c2.reshape(VOCAB, DIM),
        ctxf, qf,
    )
    out_c = (
        oc.reshape(CTX_LEN, 2, 96, 8, 128)
        .transpose(2, 4, 0, 1, 3)
        .reshape(3 * BATCH, CTX_LEN, DIM)
    )
    out_q = (
        oq.reshape(Q_LEN, 2, 96, 8, 128)
        .transpose(2, 4, 0, 1, 3)
        .reshape(3 * BATCH, Q_LEN, DIM)
    )
    return out_c, out_q
